# baseline (speedup 1.0000x reference)
import functools
import jax
import jax.numpy as jnp
from jax import lax
from jax.experimental import pallas as pl
from jax.experimental.pallas import tpu as pltpu
from jax.experimental.pallas import tpu_sc as plsc

D_MODEL = 64
SCALE = 8.0
NC = 2
NS = 16
NW = NC * NS
LANES = 16

B = 4096
L = 200
BBLK = B // NW
L2 = L // 2

_mesh = plsc.VectorSubcoreMesh(core_axis_name="c", subcore_axis_name="s")


@functools.partial(
    pl.kernel,
    out_type=jax.ShapeDtypeStruct((L, D_MODEL, B), jnp.float32),
    mesh=_mesh,
    scratch_types=[
        pltpu.VMEM((L, BBLK), jnp.int32),
        pltpu.VMEM((BBLK, D_MODEL), jnp.float32),
        pltpu.VMEM((BBLK, D_MODEL), jnp.float32),
        pltpu.VMEM((D_MODEL, BBLK), jnp.float32),
        pltpu.VMEM((D_MODEL, BBLK), jnp.float32),
        pltpu.SemaphoreType.DMA,
        pltpu.SemaphoreType.DMA,
        pltpu.SemaphoreType.DMA,
        pltpu.SemaphoreType.DMA,
    ],
    compiler_params=pltpu.CompilerParams(use_tc_tiling_on_sc=False, needs_layout_passes=False),
)
def _embed(idsT_hbm, table_hbm, out_hbm, idx_v, in0, in1, ou0, ou1,
           gs0, gs1, ws0, ws1):
    inb = (in0, in1)
    oub = (ou0, ou1)
    gsem = (gs0, gs1)
    wsem = (ws0, ws1)

    wid = lax.axis_index("s") * NC + lax.axis_index("c")
    col0 = wid * BBLK

    pltpu.sync_copy(idsT_hbm.at[:, pl.ds(col0, BBLK)], idx_v)

    iota = lax.iota(jnp.int32, LANES)
    row_idx = tuple(iota + (t * LANES) for t in range(BBLK // LANES))

    for b in range(2):
        pltpu.async_copy(table_hbm.at[idx_v.at[b]], inb[b], gsem[b])

    @pl.loop(0, L2)
    def _planes(o):
        for b in range(2):
            l = o * 2 + b

            pltpu.make_async_copy(
                table_hbm.at[idx_v.at[l]], inb[b], gsem[b]).wait()

            @pl.when(o > 0)
            def _():
                pltpu.make_async_copy(
                    oub[b], out_hbm.at[l - 2, :, pl.ds(col0, BBLK)],
                    wsem[b]).wait()

            @pl.loop(0, D_MODEL)
            def _feat(d):
                col = jnp.full((LANES,), d, jnp.int32)
                for t in range(BBLK // LANES):
                    v = plsc.load_gather(inb[b], [row_idx[t], col])
                    oub[b][d, pl.ds(t * LANES, LANES)] = v * SCALE

            @pl.when(o < L2 - 1)
            def _():
                pltpu.async_copy(
                    table_hbm.at[idx_v.at[l + 2]], inb[b], gsem[b])

            pltpu.async_copy(
                oub[b], out_hbm.at[l, :, pl.ds(col0, BBLK)], wsem[b])

    for b in range(2):
        pltpu.make_async_copy(
            oub[b], out_hbm.at[L - 2 + b, :, pl.ds(col0, BBLK)],
            wsem[b]).wait()


def kernel(input_ids, weight):
    idsT = input_ids.T
    outF = _embed(idsT, weight)
    return outF.transpose(2, 0, 1)


# speedup vs baseline: 1.4043x; 1.4043x over previous
import functools
import jax
import jax.numpy as jnp
from jax import lax
from jax.experimental import pallas as pl
from jax.experimental.pallas import tpu as pltpu
from jax.experimental.pallas import tpu_sc as plsc

D_MODEL = 64
SCALE = 8.0
NC = 2
NS = 16
NW = NC * NS
LANES = 16

B = 4096
L = 200
BBLK = B // NW
L2 = L // 2

_mesh = plsc.VectorSubcoreMesh(core_axis_name="c", subcore_axis_name="s")


@functools.partial(
    pl.kernel,
    out_type=jax.ShapeDtypeStruct((L * D_MODEL, B), jnp.float32),
    mesh=_mesh,
    scratch_types=[
        pltpu.VMEM((L, BBLK), jnp.int32),
        pltpu.VMEM((BBLK, D_MODEL), jnp.float32),
        pltpu.VMEM((BBLK, D_MODEL), jnp.float32),
        pltpu.VMEM((D_MODEL, BBLK), jnp.float32),
        pltpu.VMEM((D_MODEL, BBLK), jnp.float32),
        pltpu.SemaphoreType.DMA,
        pltpu.SemaphoreType.DMA,
        pltpu.SemaphoreType.DMA,
        pltpu.SemaphoreType.DMA,
    ],
    compiler_params=pltpu.CompilerParams(use_tc_tiling_on_sc=False, needs_layout_passes=False),
)
def _embed(idsT_hbm, table_hbm, out_hbm, idx_v, in0, in1, ou0, ou1,
           gs0, gs1, ws0, ws1):
    inb = (in0, in1)
    oub = (ou0, ou1)
    gsem = (gs0, gs1)
    wsem = (ws0, ws1)

    wid = lax.axis_index("s") * NC + lax.axis_index("c")
    col0 = wid * BBLK

    pltpu.sync_copy(idsT_hbm.at[:, pl.ds(col0, BBLK)], idx_v)

    iota = lax.iota(jnp.int32, LANES)
    row_idx = tuple(iota + (t * LANES) for t in range(BBLK // LANES))

    for b in range(2):
        pltpu.async_copy(table_hbm.at[idx_v.at[b]], inb[b], gsem[b])

    @pl.loop(0, L2)
    def _planes(o):
        for b in range(2):
            l = o * 2 + b

            pltpu.make_async_copy(
                table_hbm.at[idx_v.at[l]], inb[b], gsem[b]).wait()

            @pl.when(o > 0)
            def _():
                pltpu.make_async_copy(
                    oub[b],
                    out_hbm.at[pl.ds((l - 2) * D_MODEL, D_MODEL),
                               pl.ds(col0, BBLK)],
                    wsem[b]).wait()

            @plsc.parallel_loop(0, D_MODEL, step=1, unroll=8)
            def _feat(d):
                col = jnp.full((LANES,), d, jnp.int32)
                for t in range(BBLK // LANES):
                    v = plsc.load_gather(inb[b], [row_idx[t], col])
                    oub[b][d, pl.ds(t * LANES, LANES)] = v * SCALE

            @pl.when(o < L2 - 1)
            def _():
                pltpu.async_copy(
                    table_hbm.at[idx_v.at[l + 2]], inb[b], gsem[b])

            pltpu.async_copy(
                oub[b],
                out_hbm.at[pl.ds(l * D_MODEL, D_MODEL), pl.ds(col0, BBLK)],
                wsem[b])

    for b in range(2):
        pltpu.make_async_copy(
            oub[b],
            out_hbm.at[pl.ds((L - 2 + b) * D_MODEL, D_MODEL),
                       pl.ds(col0, BBLK)],
            wsem[b]).wait()


def kernel(input_ids, weight):
    idsT = input_ids.T
    outF = _embed(idsT, weight).reshape(L, D_MODEL, B)
    return outF.transpose(2, 0, 1)


# speedup vs baseline: 1.9220x; 1.3686x over previous
import functools
import jax
import jax.numpy as jnp
from jax import lax
from jax.experimental import pallas as pl
from jax.experimental.pallas import tpu as pltpu
from jax.experimental.pallas import tpu_sc as plsc

D_MODEL = 64
SCALE = 8.0
NC = 2
NS = 16
NW = NC * NS
LANES = 16

B = 4096
L = 200
BBLK = B // NW
CH = 4
OC = L // CH
VPF = D_MODEL // LANES

_mesh = plsc.VectorSubcoreMesh(core_axis_name="c", subcore_axis_name="s")


@functools.partial(
    pl.kernel,
    out_type=jax.ShapeDtypeStruct((B, L * D_MODEL), jnp.float32),
    mesh=_mesh,
    scratch_types=[
        pltpu.VMEM((L, BBLK), jnp.int32),
        pltpu.VMEM((BBLK, D_MODEL), jnp.float32),
        pltpu.VMEM((BBLK, D_MODEL), jnp.float32),
        pltpu.VMEM((BBLK, CH * D_MODEL), jnp.float32),
        pltpu.VMEM((BBLK, CH * D_MODEL), jnp.float32),
        pltpu.SemaphoreType.DMA,
        pltpu.SemaphoreType.DMA,
        pltpu.SemaphoreType.DMA,
        pltpu.SemaphoreType.DMA,
    ],
    compiler_params=pltpu.CompilerParams(use_tc_tiling_on_sc=False, needs_layout_passes=False),
)
def _embed(idsT_hbm, table_hbm, out_hbm, idx_v, in0, in1, ou0, ou1,
           gs0, gs1, ws0, ws1):
    inb = (in0, in1)
    oub = (ou0, ou1)
    gsem = (gs0, gs1)
    wsem = (ws0, ws1)

    wid = lax.axis_index("s") * NC + lax.axis_index("c")
    row0 = wid * BBLK

    pltpu.sync_copy(idsT_hbm.at[:, pl.ds(row0, BBLK)], idx_v)

    for b in range(2):
        pltpu.async_copy(table_hbm.at[idx_v.at[b]], inb[b], gsem[b])

    @pl.loop(0, OC // 2)
    def _chunks(oo):
        for par in range(2):
            o = oo * 2 + par

            @pl.when(oo > 0)
            def _():
                pltpu.make_async_copy(
                    oub[par],
                    out_hbm.at[pl.ds(row0, BBLK),
                               pl.ds((o - 2) * CH * D_MODEL, CH * D_MODEL)],
                    wsem[par]).wait()

            for j in range(CH):
                l = o * CH + j
                b = j % 2

                pltpu.make_async_copy(
                    table_hbm.at[idx_v.at[l]], inb[b], gsem[b]).wait()

                @plsc.parallel_loop(0, BBLK, step=1, unroll=8)
                def _rows(r):
                    for c in range(VPF):
                        v = inb[b][r, pl.ds(c * LANES, LANES)]
                        oub[par][r, pl.ds(j * D_MODEL + c * LANES, LANES)] = (
                            v * SCALE)

                @pl.when(l < L - 2)
                def _():
                    pltpu.async_copy(
                        table_hbm.at[idx_v.at[l + 2]], inb[b], gsem[b])

            pltpu.async_copy(
                oub[par],
                out_hbm.at[pl.ds(row0, BBLK),
                           pl.ds(o * CH * D_MODEL, CH * D_MODEL)],
                wsem[par])

    for par in range(2):
        o = OC - 2 + par
        pltpu.make_async_copy(
            oub[par],
            out_hbm.at[pl.ds(row0, BBLK),
                       pl.ds(o * CH * D_MODEL, CH * D_MODEL)],
            wsem[par]).wait()


def kernel(input_ids, weight):
    out2d = _embed(input_ids.T, weight)
    return out2d.reshape(B, L, D_MODEL)
